# baseline (speedup 1.0000x reference)
import functools

import jax
import jax.numpy as jnp
from jax import lax
from jax.experimental import pallas as pl
from jax.experimental.pallas import tpu as pltpu
from jax.experimental.pallas import tpu_sc as plsc

F = 26
V = 100000
E = 16
A = 8
B = 16384
NPAIR = F * (F - 1) // 2

NC, NS = 2, 16
NW = NC * NS
NROWS = B * F
ROWS_PER_W = NROWS // NW
CHUNK = 512
NCHUNK = ROWS_PER_W // CHUNK


@functools.cache
def _make_sc_gather():
    mesh = plsc.VectorSubcoreMesh(core_axis_name="c", subcore_axis_name="s")

    @functools.partial(
        pl.kernel,
        out_type=jax.ShapeDtypeStruct((NROWS * E,), jnp.float32),
        mesh=mesh,
        scratch_types=[
            pltpu.VMEM((CHUNK,), jnp.int32),
            pltpu.VMEM((CHUNK,), jnp.int32),
            pltpu.VMEM((CHUNK, 128), jnp.float32),
            pltpu.VMEM((CHUNK * E,), jnp.float32),
            pltpu.SemaphoreType.DMA,
        ],
        compiler_params=pltpu.CompilerParams(needs_layout_passes=False),
    )
    def _sc_gather(tbl_hbm, row_hbm, off_hbm, out_hbm, row_v, off_v, rows_v,
                   out_v, sem):
        wid = lax.axis_index("s") * NC + lax.axis_index("c")
        base = wid * ROWS_PER_W

        def step(i, carry):
            off = base + i * CHUNK
            pltpu.sync_copy(row_hbm.at[pl.ds(off, CHUNK)], row_v)
            pltpu.sync_copy(off_hbm.at[pl.ds(off, CHUNK)], off_v)
            pltpu.async_copy(tbl_hbm.at[row_v], rows_v, sem).wait()

            def sel(g, c2):
                lanes = lax.iota(jnp.int32, 16)
                rloc = g * 16 + lanes
                cbase = off_v[pl.ds(g * 16, 16)]
                obase = (g * 16 + lanes) * E
                for e in range(E):
                    vals = plsc.load_gather(rows_v, [rloc, cbase + e])
                    plsc.store_scatter(out_v, [obase + e], vals)
                return c2

            lax.fori_loop(0, CHUNK // 16, sel, 0)
            pltpu.sync_copy(out_v, out_hbm.at[pl.ds(off * E, CHUNK * E)])
            return carry

        lax.fori_loop(0, NCHUNK, step, 0)

    return _sc_gather


BB = 256
NP13 = 13
PW = 512
IW = NP13 * PW
SW = 416


def _afm_body(x_ref, emb_ref, wt_ref, abt_ref, ht_ref, e32_ref, es_ref,
              mask_ref, pp_ref, ww_ref, wb_ref, out_ref):
    e2 = emb_ref[...]
    zpad = jnp.zeros((BB, PW - F * E), jnp.float32)
    pieces = []
    for p in range(NP13):
        d = (p + 1) * E
        rot = jnp.concatenate([e2[:, d:], e2[:, :d]], axis=1)
        pieces.append(jnp.concatenate([e2 * rot, zpad], axis=1))
    ifull = jnp.concatenate(pieces, axis=1)

    score_gs = []
    for g in range(2 * NP13):
        sl = ifull[:, 256 * g:256 * (g + 1)]
        att_g = jnp.maximum(
            jnp.dot(sl, wt_ref[...], preferred_element_type=jnp.float32)
            + abt_ref[...], 0.0)
        score_gs.append(
            jnp.dot(att_g, ht_ref[...], preferred_element_type=jnp.float32))
    score = jnp.concatenate(score_gs, axis=1) + mask_ref[...]

    m = jnp.max(score, axis=1, keepdims=True)
    ex = jnp.exp(score - m)
    w = ex / jnp.sum(ex, axis=1, keepdims=True)

    att_out = jnp.zeros((BB, E), jnp.float32)
    for p in range(NP13):
        wexp_p = jnp.dot(w[:, 32 * p:32 * p + 32], e32_ref[...],
                         preferred_element_type=jnp.float32)
        u_p = ifull[:, PW * p:PW * (p + 1)] * wexp_p
        att_out = att_out + jnp.dot(u_p, es_ref[...],
                                    preferred_element_type=jnp.float32)
    afm = jnp.sum(att_out * pp_ref[...], axis=1)
    wide = jnp.maximum(
        jnp.sum(x_ref[...] * ww_ref[...], axis=1) + wb_ref[0, 0], 0.0)
    out_ref[...] = jax.nn.sigmoid(wide + afm).reshape(BB, 1)


def _afm_tc(X, emb2, wt, abt, ht, e32, es, mask, pp_row, ww, wb2):
    nblk = B // BB
    full = lambda shp: pl.BlockSpec(shp, lambda i: tuple(0 for _ in shp))
    return pl.pallas_call(
        _afm_body,
        grid=(nblk,),
        in_specs=[
            pl.BlockSpec((BB, F), lambda i: (i, 0)),
            pl.BlockSpec((BB, F * E), lambda i: (i, 0)),
            full((256, 128)),
            full((1, 128)),
            full((128, E)),
            full((32, PW)),
            full((PW, E)),
            full((1, SW)),
            full((1, E)),
            full((1, F)),
            full((1, 1)),
        ],
        out_specs=pl.BlockSpec((BB, 1), lambda i: (i, 0)),
        out_shape=jax.ShapeDtypeStruct((B, 1), jnp.float32),
    )(X, emb2, wt, abt, ht, e32, es, mask, pp_row, ww, wb2)


def kernel(X, tables, attention_W, attention_b, projection_h, projection_p,
           wide_W, wide_b):
    idx = (X.astype(jnp.int32)
           + (jnp.arange(F, dtype=jnp.int32) * V)[None, :]).reshape(-1)
    row = idx // 8
    off16 = (idx % 8) * E
    tbl128 = tables.reshape(F * V // 8, 128)
    emb_flat = _make_sc_gather()(tbl128, row, off16)
    emb2 = emb_flat.reshape(B, F * E)

    eye16 = jnp.eye(E, dtype=jnp.float32)
    wt = jnp.kron(eye16, attention_W)
    abt = jnp.tile(attention_b.reshape(1, A), (1, E))
    ht = jnp.kron(eye16, projection_h.reshape(A, 1))
    lane = jnp.arange(PW)
    e32 = ((lane[None, :] // E == jnp.arange(32)[:, None])
           & (lane[None, :] < F * E)).astype(jnp.float32)
    es = (lane[:, None] % E == jnp.arange(E)[None, :]).astype(jnp.float32)
    sl = jnp.arange(SW)
    valid = (sl % 32 < F) & ((sl // 32 < NP13 - 1) | (sl % 32 < NP13))
    mask = jnp.where(valid, 0.0, -1e30).astype(jnp.float32).reshape(1, SW)

    out2 = _afm_tc(X, emb2, wt, abt, ht, e32, es, mask,
                   projection_p.reshape(1, E), wide_W, wide_b.reshape(1, 1))
    return out2.reshape(B)


# speedup vs baseline: 2.5245x; 2.5245x over previous
import functools

import jax
import jax.numpy as jnp
from jax import lax
from jax.experimental import pallas as pl
from jax.experimental.pallas import tpu as pltpu
from jax.experimental.pallas import tpu_sc as plsc

F = 26
V = 100000
E = 16
A = 8
B = 16384
NPAIR = F * (F - 1) // 2

NC, NS = 2, 16
NW = NC * NS
NROWS = B * F
NCOMP = F * E
COMP_PER_W = NCOMP // NW
BCH = 8192
NBCH = B // BCH


@functools.cache
def _make_sc_gather():
    mesh = plsc.VectorSubcoreMesh(core_axis_name="c", subcore_axis_name="s")

    @functools.partial(
        pl.kernel,
        out_type=jax.ShapeDtypeStruct((NCOMP, B), jnp.float32),
        mesh=mesh,
        scratch_types=[
            pltpu.VMEM((V,), jnp.float32),
            pltpu.VMEM((BCH,), jnp.int32),
            pltpu.VMEM((BCH,), jnp.float32),
            pltpu.SemaphoreType.DMA,
        ],
        compiler_params=pltpu.CompilerParams(needs_layout_passes=False),
    )
    def _sc_gather(tbl_hbm, xt_hbm, out_hbm, tbl_v, idx_v, out_v, sem):
        wid = lax.axis_index("s") * NC + lax.axis_index("c")
        r0 = wid * COMP_PER_W

        def row_loop(k, c):
            r = r0 + k
            f = r // E
            pltpu.sync_copy(tbl_hbm.at[f, r % E], tbl_v)

            def chunk(j, c2):
                pltpu.sync_copy(xt_hbm.at[f, pl.ds(j * BCH, BCH)], idx_v)

                def grp(g, c3):
                    iv = idx_v[pl.ds(g * 16, 16)]
                    out_v[pl.ds(g * 16, 16)] = plsc.load_gather(tbl_v, [iv])
                    return c3

                lax.fori_loop(0, BCH // 16, grp, 0)
                pltpu.sync_copy(out_v, out_hbm.at[r, pl.ds(j * BCH, BCH)])
                return c2

            lax.fori_loop(0, NBCH, chunk, 0)
            return c

        lax.fori_loop(0, COMP_PER_W, row_loop, 0)

    return _sc_gather


BB = 256
NP13 = 13
PW = 512
IW = NP13 * PW
SW = 416


def _afm_body(x_ref, emb_ref, wt_ref, abt_ref, ht_ref, e32_ref, es_ref,
              mask_ref, pp_ref, ww_ref, wb_ref, out_ref):
    e2 = emb_ref[...]
    zpad = jnp.zeros((BB, PW - F * E), jnp.float32)
    pieces = []
    for p in range(NP13):
        d = (p + 1) * E
        rot = jnp.concatenate([e2[:, d:], e2[:, :d]], axis=1)
        pieces.append(jnp.concatenate([e2 * rot, zpad], axis=1))
    ifull = jnp.concatenate(pieces, axis=1)

    score_gs = []
    for g in range(2 * NP13):
        sl = ifull[:, 256 * g:256 * (g + 1)]
        att_g = jnp.maximum(
            jnp.dot(sl, wt_ref[...], preferred_element_type=jnp.float32)
            + abt_ref[...], 0.0)
        score_gs.append(
            jnp.dot(att_g, ht_ref[...], preferred_element_type=jnp.float32))
    score = jnp.concatenate(score_gs, axis=1) + mask_ref[...]

    m = jnp.max(score, axis=1, keepdims=True)
    ex = jnp.exp(score - m)
    w = ex / jnp.sum(ex, axis=1, keepdims=True)

    att_out = jnp.zeros((BB, E), jnp.float32)
    for p in range(NP13):
        wexp_p = jnp.dot(w[:, 32 * p:32 * p + 32], e32_ref[...],
                         preferred_element_type=jnp.float32)
        u_p = ifull[:, PW * p:PW * (p + 1)] * wexp_p
        att_out = att_out + jnp.dot(u_p, es_ref[...],
                                    preferred_element_type=jnp.float32)
    afm = jnp.sum(att_out * pp_ref[...], axis=1)
    wide = jnp.maximum(
        jnp.sum(x_ref[...] * ww_ref[...], axis=1) + wb_ref[0, 0], 0.0)
    out_ref[...] = jax.nn.sigmoid(wide + afm).reshape(BB, 1)


def _afm_tc(X, emb2, wt, abt, ht, e32, es, mask, pp_row, ww, wb2):
    nblk = B // BB
    full = lambda shp: pl.BlockSpec(shp, lambda i: tuple(0 for _ in shp))
    return pl.pallas_call(
        _afm_body,
        grid=(nblk,),
        in_specs=[
            pl.BlockSpec((BB, F), lambda i: (i, 0)),
            pl.BlockSpec((BB, F * E), lambda i: (i, 0)),
            full((256, 128)),
            full((1, 128)),
            full((128, E)),
            full((32, PW)),
            full((PW, E)),
            full((1, SW)),
            full((1, E)),
            full((1, F)),
            full((1, 1)),
        ],
        out_specs=pl.BlockSpec((BB, 1), lambda i: (i, 0)),
        out_shape=jax.ShapeDtypeStruct((B, 1), jnp.float32),
    )(X, emb2, wt, abt, ht, e32, es, mask, pp_row, ww, wb2)


def kernel(X, tables, attention_W, attention_b, projection_h, projection_p,
           wide_W, wide_b):
    XT = X.astype(jnp.int32).T
    tblT = tables.transpose(0, 2, 1)
    embT = _make_sc_gather()(tblT, XT)
    emb2 = embT.T

    eye16 = jnp.eye(E, dtype=jnp.float32)
    wt = jnp.kron(eye16, attention_W)
    abt = jnp.tile(attention_b.reshape(1, A), (1, E))
    ht = jnp.kron(eye16, projection_h.reshape(A, 1))
    lane = jnp.arange(PW)
    e32 = ((lane[None, :] // E == jnp.arange(32)[:, None])
           & (lane[None, :] < F * E)).astype(jnp.float32)
    es = (lane[:, None] % E == jnp.arange(E)[None, :]).astype(jnp.float32)
    sl = jnp.arange(SW)
    valid = (sl % 32 < F) & ((sl // 32 < NP13 - 1) | (sl % 32 < NP13))
    mask = jnp.where(valid, 0.0, -1e30).astype(jnp.float32).reshape(1, SW)

    out2 = _afm_tc(X, emb2, wt, abt, ht, e32, es, mask,
                   projection_p.reshape(1, E), wide_W, wide_b.reshape(1, 1))
    return out2.reshape(B)


# speedup vs baseline: 2.7341x; 1.0830x over previous
import functools

import jax
import jax.numpy as jnp
from jax import lax
from jax.experimental import pallas as pl
from jax.experimental.pallas import tpu as pltpu
from jax.experimental.pallas import tpu_sc as plsc

F = 26
V = 100000
E = 16
A = 8
B = 16384
NPAIR = F * (F - 1) // 2

NC, NS = 2, 16
NW = NC * NS
NROWS = B * F
NCOMP = F * E
COMP_PER_W = NCOMP // NW
BCH = 8192
NBCH = B // BCH


@functools.cache
def _make_sc_gather(nb):
    mesh = plsc.VectorSubcoreMesh(core_axis_name="c", subcore_axis_name="s")
    bch = min(nb, BCH)
    nbch = nb // bch

    @functools.partial(
        pl.kernel,
        out_type=jax.ShapeDtypeStruct((NCOMP, nb), jnp.float32),
        mesh=mesh,
        scratch_types=[
            pltpu.VMEM((V,), jnp.float32),
            pltpu.VMEM((bch,), jnp.int32),
            pltpu.VMEM((bch,), jnp.float32),
            pltpu.SemaphoreType.DMA,
        ],
        compiler_params=pltpu.CompilerParams(needs_layout_passes=False),
    )
    def _sc_gather(tbl_hbm, xt_hbm, out_hbm, tbl_v, idx_v, out_v, sem):
        wid = lax.axis_index("s") * NC + lax.axis_index("c")
        r0 = wid * COMP_PER_W

        def row_loop(k, c):
            r = r0 + k
            f = r // E
            pltpu.sync_copy(tbl_hbm.at[f, r % E], tbl_v)

            def chunk(j, c2):
                pltpu.sync_copy(xt_hbm.at[f, pl.ds(j * bch, bch)], idx_v)

                def grp(g, c3):
                    iv = idx_v[pl.ds(g * 16, 16)]
                    out_v[pl.ds(g * 16, 16)] = plsc.load_gather(tbl_v, [iv])
                    return c3

                lax.fori_loop(0, bch // 16, grp, 0)
                pltpu.sync_copy(out_v, out_hbm.at[r, pl.ds(j * bch, bch)])
                return c2

            lax.fori_loop(0, nbch, chunk, 0)
            return c

        lax.fori_loop(0, COMP_PER_W, row_loop, 0)

    return _sc_gather


BB = 256
NP13 = 13
PW = 512
IW = NP13 * PW
SW = 416


def _afm_body(x_ref, emb_ref, wt_ref, abt_ref, ht_ref, e32_ref, es_ref,
              mask_ref, pp_ref, ww_ref, wb_ref, out_ref):
    e2 = emb_ref[...]
    zpad = jnp.zeros((BB, PW - F * E), jnp.float32)
    pieces = []
    for p in range(NP13):
        d = (p + 1) * E
        rot = jnp.concatenate([e2[:, d:], e2[:, :d]], axis=1)
        pieces.append(jnp.concatenate([e2 * rot, zpad], axis=1))
    ifull = jnp.concatenate(pieces, axis=1)

    score_gs = []
    for g in range(2 * NP13):
        sl = ifull[:, 256 * g:256 * (g + 1)]
        att_g = jnp.maximum(
            jnp.dot(sl, wt_ref[...], preferred_element_type=jnp.float32)
            + abt_ref[...], 0.0)
        score_gs.append(
            jnp.dot(att_g, ht_ref[...], preferred_element_type=jnp.float32))
    score = jnp.concatenate(score_gs, axis=1) + mask_ref[...]

    m = jnp.max(score, axis=1, keepdims=True)
    ex = jnp.exp(score - m)
    w = ex / jnp.sum(ex, axis=1, keepdims=True)

    att_out = jnp.zeros((BB, E), jnp.float32)
    for p in range(NP13):
        wexp_p = jnp.dot(w[:, 32 * p:32 * p + 32], e32_ref[...],
                         preferred_element_type=jnp.float32)
        u_p = ifull[:, PW * p:PW * (p + 1)] * wexp_p
        att_out = att_out + jnp.dot(u_p, es_ref[...],
                                    preferred_element_type=jnp.float32)
    afm = jnp.sum(att_out * pp_ref[...], axis=1)
    wide = jnp.maximum(
        jnp.sum(x_ref[...] * ww_ref[...], axis=1) + wb_ref[0, 0], 0.0)
    out_ref[...] = jax.nn.sigmoid(wide + afm).reshape(BB, 1)


def _afm_tc(X, emb2, wt, abt, ht, e32, es, mask, pp_row, ww, wb2):
    nb = X.shape[0]
    nblk = nb // BB
    full = lambda shp: pl.BlockSpec(shp, lambda i: tuple(0 for _ in shp))
    return pl.pallas_call(
        _afm_body,
        grid=(nblk,),
        in_specs=[
            pl.BlockSpec((BB, F), lambda i: (i, 0)),
            pl.BlockSpec((BB, F * E), lambda i: (i, 0)),
            full((256, 128)),
            full((1, 128)),
            full((128, E)),
            full((32, PW)),
            full((PW, E)),
            full((1, SW)),
            full((1, E)),
            full((1, F)),
            full((1, 1)),
        ],
        out_specs=pl.BlockSpec((BB, 1), lambda i: (i, 0)),
        out_shape=jax.ShapeDtypeStruct((nb, 1), jnp.float32),
    )(X, emb2, wt, abt, ht, e32, es, mask, pp_row, ww, wb2)


def kernel(X, tables, attention_W, attention_b, projection_h, projection_p,
           wide_W, wide_b):
    XT = X.astype(jnp.int32).T
    tblT = tables.transpose(0, 2, 1)

    eye16 = jnp.eye(E, dtype=jnp.float32)
    wt = jnp.kron(eye16, attention_W)
    abt = jnp.tile(attention_b.reshape(1, A), (1, E))
    ht = jnp.kron(eye16, projection_h.reshape(A, 1))
    lane = jnp.arange(PW)
    e32 = ((lane[None, :] // E == jnp.arange(32)[:, None])
           & (lane[None, :] < F * E)).astype(jnp.float32)
    es = (lane[:, None] % E == jnp.arange(E)[None, :]).astype(jnp.float32)
    sl = jnp.arange(SW)
    valid = (sl % 32 < F) & ((sl // 32 < NP13 - 1) | (sl % 32 < NP13))
    mask = jnp.where(valid, 0.0, -1e30).astype(jnp.float32).reshape(1, SW)

    nsplit = 2
    hb = B // nsplit
    outs = []
    for h in range(nsplit):
        xt_h = XT[:, h * hb:(h + 1) * hb]
        embT_h = _make_sc_gather(hb)(tblT, xt_h)
        out_h = _afm_tc(X[h * hb:(h + 1) * hb], embT_h.T, wt, abt, ht, e32,
                        es, mask, projection_p.reshape(1, E), wide_W,
                        wide_b.reshape(1, 1))
        outs.append(out_h)
    return jnp.concatenate(outs, axis=0).reshape(B)
